# baseline (speedup 1.0000x reference)
import functools

import jax
import jax.numpy as jnp
from jax.experimental import pallas as pl
from jax.experimental.pallas import tpu as pltpu

N_LAYER = 6
B, S, V, D, H = 2, 1024, 32000, 768, 12
T = S + 1
VP = 32768
VBLK = 1024
NV = VP // VBLK
RBLK = 1024
GBLK = 128


def _gather_body(idx_ref, we_ref, out_ref, sem):
    g = pl.program_id(0)
    base = g * GBLK
    for mi in range(GBLK):
        row = idx_ref[base + mi]
        pltpu.make_async_copy(
            we_ref.at[pl.ds(row, 1), :], out_ref.at[pl.ds(mi, 1), :], sem
        ).start()
    for mi in range(GBLK):
        pltpu.make_async_copy(
            we_ref.at[pl.ds(0, 1), :], out_ref.at[pl.ds(mi, 1), :], sem
        ).wait()


def _k_body(p_ref, pmT_ref, wkT_ref, wq_ref, k_ref):
    xiT = jnp.dot(wkT_ref[0], pmT_ref[...],
                  preferred_element_type=jnp.float32)
    xj = jnp.dot(p_ref[...], wq_ref[0],
                 preferred_element_type=jnp.float32)
    k_ref[0] = jnp.dot(xj, xiT, preferred_element_type=jnp.float32)


def _vocab_body(f_ref, weT_ref, we_ref, e_ref, out_ref, num_ref, den_ref):
    v = pl.program_id(1)

    @pl.when(v == 0)
    def _():
        num_ref[...] = jnp.zeros_like(num_ref)
        den_ref[...] = jnp.zeros_like(den_ref)

    logits = jnp.dot(f_ref[...], weT_ref[...],
                     preferred_element_type=jnp.float32)
    ex = jnp.exp(logits)
    num_ref[...] += jnp.dot(ex, we_ref[...],
                            preferred_element_type=jnp.float32)
    den_ref[...] += jnp.sum(ex, axis=1, keepdims=True)

    @pl.when(v == NV - 1)
    def _():
        den = den_ref[:, :1] - float(VP - V) + 1e-8
        out_ref[...] = e_ref[...] - num_ref[...] / den


def _head_body(diff_ref, k_ref, wv_ref, f_ref, a_ref, b_ref, out_ref, acc_ref):
    h = pl.program_id(1)

    @pl.when(h == 0)
    def _():
        db = jnp.sum(diff_ref[0], axis=0, keepdims=True) * b_ref[0, 0]
        acc_ref[...] = jnp.broadcast_to(db, acc_ref.shape)

    tmp = jnp.dot(diff_ref[0], wv_ref[0],
                  preferred_element_type=jnp.float32) * a_ref[0, h]
    acc_ref[...] += jnp.dot(k_ref[0], tmp, preferred_element_type=jnp.float32)

    @pl.when(h == H - 1)
    def _():
        out_ref[0] = f_ref[0] + acc_ref[...] * (1.0 / S)


def _logits_body(fl_ref, weT_ref, out_ref):
    out_ref[...] = jnp.dot(fl_ref[...], weT_ref[...],
                           preferred_element_type=jnp.float32)


def kernel(idx, W_e, W_p, W_k, W_q, W_v, A_LR, B_LR):
    f32 = jnp.float32
    p = W_p[:T]
    pmT = p[:S].T
    W_kT = W_k.transpose(0, 2, 1)
    We_pad = jnp.pad(W_e, ((0, VP - V), (0, 0)))
    WeT = We_pad.T
    A2 = A_LR.reshape(1, H).astype(f32)
    B2 = B_LR.reshape(1, 1).astype(f32)

    e = pl.pallas_call(
        _gather_body,
        out_shape=jax.ShapeDtypeStruct((B * S, D), f32),
        grid_spec=pltpu.PrefetchScalarGridSpec(
            num_scalar_prefetch=1,
            grid=(B * S // GBLK,),
            in_specs=[pl.BlockSpec(memory_space=pl.ANY)],
            out_specs=pl.BlockSpec((GBLK, D), lambda g, i: (g, 0)),
            scratch_shapes=[pltpu.SemaphoreType.DMA],
        ),
        compiler_params=pltpu.CompilerParams(
            dimension_semantics=("arbitrary",)),
        name="pgd_gather",
    )(idx.reshape(-1).astype(jnp.int32), W_e)

    K = pl.pallas_call(
        _k_body,
        out_shape=jax.ShapeDtypeStruct((H, T, S), f32),
        grid=(H,),
        in_specs=[
            pl.BlockSpec((T, D), lambda h: (0, 0)),
            pl.BlockSpec((D, S), lambda h: (0, 0)),
            pl.BlockSpec((1, D, D), lambda h: (h, 0, 0)),
            pl.BlockSpec((1, D, D), lambda h: (h, 0, 0)),
        ],
        out_specs=pl.BlockSpec((1, T, S), lambda h: (h, 0, 0)),
        compiler_params=pltpu.CompilerParams(
            dimension_semantics=("parallel",),
            vmem_limit_bytes=56 * 1024 * 1024),
        name="pgd_kmat",
    )(p, pmT, W_kT, W_q)

    vocab_call = pl.pallas_call(
        _vocab_body,
        out_shape=jax.ShapeDtypeStruct((B * S, D), f32),
        grid=(B * S // RBLK, NV),
        in_specs=[
            pl.BlockSpec((RBLK, D), lambda r, v: (r, 0)),
            pl.BlockSpec((D, VBLK), lambda r, v: (0, v)),
            pl.BlockSpec((VBLK, D), lambda r, v: (v, 0)),
            pl.BlockSpec((RBLK, D), lambda r, v: (r, 0)),
        ],
        out_specs=pl.BlockSpec((RBLK, D), lambda r, v: (r, 0)),
        scratch_shapes=[
            pltpu.VMEM((RBLK, D), f32),
            pltpu.VMEM((RBLK, 128), f32),
        ],
        compiler_params=pltpu.CompilerParams(
            dimension_semantics=("parallel", "arbitrary"),
            vmem_limit_bytes=56 * 1024 * 1024),
        name="pgd_vocab",
    )

    head_call = pl.pallas_call(
        _head_body,
        out_shape=jax.ShapeDtypeStruct((B, T, D), f32),
        grid=(B, H),
        in_specs=[
            pl.BlockSpec((1, S, D), lambda b, h: (b, 0, 0)),
            pl.BlockSpec((1, T, S), lambda b, h: (h, 0, 0)),
            pl.BlockSpec((1, D, D), lambda b, h: (h, 0, 0)),
            pl.BlockSpec((1, T, D), lambda b, h: (b, 0, 0)),
            pl.BlockSpec(memory_space=pltpu.SMEM),
            pl.BlockSpec(memory_space=pltpu.SMEM),
        ],
        out_specs=pl.BlockSpec((1, T, D), lambda b, h: (b, 0, 0)),
        scratch_shapes=[pltpu.VMEM((T, D), f32)],
        compiler_params=pltpu.CompilerParams(
            dimension_semantics=("parallel", "arbitrary"),
            vmem_limit_bytes=56 * 1024 * 1024),
        name="pgd_head",
    )

    f = jnp.zeros((B, T, D), f32)
    for _ in range(N_LAYER):
        fs = f[:, :S].reshape(B * S, D)
        diff = vocab_call(fs, WeT, We_pad, e)
        f = head_call(diff.reshape(B, S, D), K, W_v, f, A2, B2)

    fl = jnp.pad(f[:, S], ((0, 8 - B), (0, 0)))
    LBLK = 4096
    lg = pl.pallas_call(
        _logits_body,
        out_shape=jax.ShapeDtypeStruct((8, VP), f32),
        grid=(VP // LBLK,),
        in_specs=[
            pl.BlockSpec((8, D), lambda v: (0, 0)),
            pl.BlockSpec((D, LBLK), lambda v: (0, v)),
        ],
        out_specs=pl.BlockSpec((8, LBLK), lambda v: (0, v)),
        compiler_params=pltpu.CompilerParams(
            dimension_semantics=("arbitrary",),
            vmem_limit_bytes=56 * 1024 * 1024),
        name="pgd_logits",
    )(fl, WeT)
    return lg[:B, :V]


# speedup vs baseline: 1.1758x; 1.1758x over previous
import jax
import jax.numpy as jnp
from jax import lax
from jax.experimental import pallas as pl
from jax.experimental.pallas import tpu as pltpu

N_LAYER = 6
B, S, V, D, H = 2, 1024, 32000, 768, 12
T = S + 1
VBLK = 1280
NV = V // VBLK
GBLK = 128

_CONTRACT_LAST = (((1,), (1,)), ((), ()))


def _gather_body(idx_ref, we_ref, out_ref, sem):
    g = pl.program_id(0)
    base = g * GBLK
    for mi in range(GBLK):
        row = idx_ref[base + mi]
        pltpu.make_async_copy(
            we_ref.at[pl.ds(row, 1), :], out_ref.at[0, pl.ds(mi, 1), :], sem
        ).start()
    for mi in range(GBLK):
        pltpu.make_async_copy(
            we_ref.at[pl.ds(0, 1), :], out_ref.at[0, pl.ds(mi, 1), :], sem
        ).wait()


def _k_body(p_ref, wk_ref, wq_ref, k_ref):
    xi = jnp.dot(p_ref[:S], wk_ref[0], preferred_element_type=jnp.float32)
    xj = jnp.dot(p_ref[...], wq_ref[0], preferred_element_type=jnp.float32)
    k_ref[0] = lax.dot_general(xj, xi, _CONTRACT_LAST,
                               preferred_element_type=jnp.float32)


def _vocab_body(f_ref, we_ref, e_ref, out_ref, num_ref, den_ref):
    v = pl.program_id(1)

    @pl.when(v == 0)
    def _():
        num_ref[...] = jnp.zeros_like(num_ref)
        den_ref[...] = jnp.zeros_like(den_ref)

    logits = lax.dot_general(f_ref[0, :S], we_ref[...], _CONTRACT_LAST,
                             preferred_element_type=jnp.float32)
    ex = jnp.exp(logits)
    num_ref[...] += jnp.dot(ex, we_ref[...],
                            preferred_element_type=jnp.float32)
    den_ref[...] += jnp.sum(ex, axis=1, keepdims=True)

    @pl.when(v == NV - 1)
    def _():
        den = den_ref[:, :1] + 1e-8
        out_ref[0] = e_ref[0] - num_ref[...] / den


def _head_body(diff_ref, k_ref, wv_ref, f_ref, a_ref, b_ref, out_ref, acc_ref):
    h = pl.program_id(1)

    @pl.when(h == 0)
    def _():
        db = jnp.sum(diff_ref[0], axis=0, keepdims=True) * b_ref[0, 0]
        acc_ref[...] = jnp.broadcast_to(db, acc_ref.shape)

    tmp = jnp.dot(diff_ref[0], wv_ref[0],
                  preferred_element_type=jnp.float32) * a_ref[0, h]
    acc_ref[...] += jnp.dot(k_ref[0], tmp, preferred_element_type=jnp.float32)

    @pl.when(h == H - 1)
    def _():
        out_ref[0] = f_ref[0] + acc_ref[...] * (1.0 / S)


def _logits_body(fl_ref, we_ref, out_ref):
    out_ref[...] = lax.dot_general(fl_ref[...], we_ref[...], _CONTRACT_LAST,
                                   preferred_element_type=jnp.float32)


def kernel(idx, W_e, W_p, W_k, W_q, W_v, A_LR, B_LR):
    f32 = jnp.float32
    p = W_p[:T]
    A2 = A_LR.reshape(1, H).astype(f32)
    B2 = B_LR.reshape(1, 1).astype(f32)

    e = pl.pallas_call(
        _gather_body,
        out_shape=jax.ShapeDtypeStruct((B, S, D), f32),
        grid_spec=pltpu.PrefetchScalarGridSpec(
            num_scalar_prefetch=1,
            grid=(B * S // GBLK,),
            in_specs=[pl.BlockSpec(memory_space=pl.ANY)],
            out_specs=pl.BlockSpec(
                (1, GBLK, D), lambda g, i: (g // (S // GBLK), g % (S // GBLK), 0)),
            scratch_shapes=[pltpu.SemaphoreType.DMA],
        ),
        compiler_params=pltpu.CompilerParams(
            dimension_semantics=("arbitrary",)),
        name="pgd_gather",
    )(idx.reshape(-1).astype(jnp.int32), W_e)

    K = pl.pallas_call(
        _k_body,
        out_shape=jax.ShapeDtypeStruct((H, T, S), f32),
        grid=(H,),
        in_specs=[
            pl.BlockSpec((T, D), lambda h: (0, 0)),
            pl.BlockSpec((1, D, D), lambda h: (h, 0, 0)),
            pl.BlockSpec((1, D, D), lambda h: (h, 0, 0)),
        ],
        out_specs=pl.BlockSpec((1, T, S), lambda h: (h, 0, 0)),
        compiler_params=pltpu.CompilerParams(
            dimension_semantics=("arbitrary",),
            vmem_limit_bytes=56 * 1024 * 1024),
        name="pgd_kmat",
    )(p, W_k, W_q)

    vocab_call = pl.pallas_call(
        _vocab_body,
        out_shape=jax.ShapeDtypeStruct((B, S, D), f32),
        grid=(B, NV),
        in_specs=[
            pl.BlockSpec((1, S, D), lambda r, v: (r, 0, 0)),
            pl.BlockSpec((VBLK, D), lambda r, v: (v, 0)),
            pl.BlockSpec((1, S, D), lambda r, v: (r, 0, 0)),
        ],
        out_specs=pl.BlockSpec((1, S, D), lambda r, v: (r, 0, 0)),
        scratch_shapes=[
            pltpu.VMEM((S, D), f32),
            pltpu.VMEM((S, 128), f32),
        ],
        compiler_params=pltpu.CompilerParams(
            dimension_semantics=("parallel", "arbitrary"),
            vmem_limit_bytes=56 * 1024 * 1024),
        name="pgd_vocab",
    )

    head_call = pl.pallas_call(
        _head_body,
        out_shape=jax.ShapeDtypeStruct((B, T, D), f32),
        grid=(B, H),
        in_specs=[
            pl.BlockSpec((1, S, D), lambda b, h: (b, 0, 0)),
            pl.BlockSpec((1, T, S), lambda b, h: (h, 0, 0)),
            pl.BlockSpec((1, D, D), lambda b, h: (h, 0, 0)),
            pl.BlockSpec((1, T, D), lambda b, h: (b, 0, 0)),
            pl.BlockSpec(memory_space=pltpu.SMEM),
            pl.BlockSpec(memory_space=pltpu.SMEM),
        ],
        out_specs=pl.BlockSpec((1, T, D), lambda b, h: (b, 0, 0)),
        scratch_shapes=[pltpu.VMEM((T, D), f32)],
        compiler_params=pltpu.CompilerParams(
            dimension_semantics=("parallel", "arbitrary"),
            vmem_limit_bytes=56 * 1024 * 1024),
        name="pgd_head",
    )

    f = jnp.zeros((B, T, D), f32)
    for _ in range(N_LAYER):
        diff = vocab_call(f, W_e, e)
        f = head_call(diff, K, W_v, f, A2, B2)

    fl = jnp.pad(f[:, S], ((0, 8 - B), (0, 0)))
    LBLK = 3200
    lg = pl.pallas_call(
        _logits_body,
        out_shape=jax.ShapeDtypeStruct((8, V), f32),
        grid=(V // LBLK,),
        in_specs=[
            pl.BlockSpec((8, D), lambda v: (0, 0)),
            pl.BlockSpec((LBLK, D), lambda v: (v, 0)),
        ],
        out_specs=pl.BlockSpec((8, LBLK), lambda v: (0, v)),
        compiler_params=pltpu.CompilerParams(
            dimension_semantics=("arbitrary",),
            vmem_limit_bytes=56 * 1024 * 1024),
        name="pgd_logits",
    )(fl, W_e)
    return lg[:B, :V]
